# baseline (speedup 1.0000x reference)
import functools

import jax
import jax.numpy as jnp
from jax import lax
from jax.experimental import pallas as pl
from jax.experimental.pallas import tpu as pltpu
from jax.experimental.pallas import tpu_sc as plsc

R = 65536
N = 128
NB = N + 1
OUT = 2 * NB
HIST_PAD = 0.01
EPS = 1e-5

NC = 2
NS = 16
L = 16
NW = NC * NS
RPT = R // NW
CH = 64
NG = CH // L
NCHUNK = RPT // CH
NPAIR = NCHUNK // 2


def _sc_body(w_hbm, eb_hbm, near_hbm, far_hbm, out_hbm,
             w_v0, w_v1, eb_v0, eb_v1, near_v, far_v, cumw_v, cdf_v, hist_v,
             out_v0, out_v1, sem_in0, sem_in1, sem_out0, sem_out1):
    cid = lax.axis_index("c")
    sid = lax.axis_index("s")
    wid = sid * NC + cid
    lane = lax.iota(jnp.int32, L)
    ones_i = jnp.ones((L,), jnp.int32)
    zeros_i = jnp.zeros((L,), jnp.int32)
    zeros_f = jnp.zeros((L,), jnp.float32)

    @plsc.parallel_loop(0, 2 * (NB + 1), unroll=8, carry=lane)
    def _z(i, idx):
        plsc.store_scatter(hist_v, [idx], zeros_i)
        return idx + L

    def do_group_pair(c, gp, w_v, eb_v, out_v):
        st = []
        for k in (0, 1):
            g = 2 * gp + k
            gl = g * L + lane
            nfidx = c * CH + gl
            near = plsc.load_gather(near_v, [nfidx])
            fnd = plsc.load_gather(far_v, [nfidx]) - near
            st.append({
                "gl": gl, "ebbase": gl * NB, "outbase": gl * OUT,
                "near": near, "fnd": fnd,
                "cumw0": lane + k * (L * N),
                "cdf0": lane + k * (L * NB),
                "hist0": lane + k * (L * (NB + 1)),
            })

        @plsc.parallel_loop(0, N, unroll=4, carry=tuple(
            (s["gl"] * N, s["cumw0"], zeros_f) for s in st))
        def apair(m, carry):
            nxt = []
            for k in (0, 1):
                widx, cumidx, cum = carry[k]
                cum = cum + (plsc.load_gather(w_v, [widx]) + HIST_PAD)
                plsc.store_scatter(cumw_v, [cumidx], cum)
                nxt.append((widx + 1, cumidx + L, cum))
            return tuple(nxt)
        padstep = []
        inv = []
        for k in (0, 1):
            s_k = apair[k][2]
            pad = jnp.maximum(EPS - s_k, 0.0)
            padstep.append(pad * (1.0 / N))
            inv.append(1.0 / (s_k + pad))

        for k in (0, 1):
            s = st[k]
            plsc.store_scatter(cdf_v, [s["cdf0"]], zeros_f)
            plsc.addupdate_scatter(hist_v, [s["hist0"]], ones_i)
            eb0v = plsc.load_gather(eb_v, [s["ebbase"]])
            plsc.store_scatter(out_v, [s["outbase"]],
                               s["near"] + eb0v * s["fnd"])

        @plsc.parallel_loop(1, NB, unroll=4, carry=tuple(
            (s["cumw0"], s["cdf0"] + L, s["ebbase"] + 1, s["outbase"] + 1,
             padstep[k]) for k, s in enumerate(st)))
        def bpair(m, carry):
            nxt = []
            for k in (0, 1):
                cumidx, cdfidx, ebidx, outidx, mpad = carry[k]
                cw = plsc.load_gather(cumw_v, [cumidx])
                cc = jnp.minimum(1.0, (cw + mpad) * inv[k])
                plsc.store_scatter(cdf_v, [cdfidx], cc)
                x = cc * jnp.float32(NB) - 0.5
                xi = x.astype(jnp.int32)
                b = xi + jnp.where(xi.astype(jnp.float32) < x, ones_i, zeros_i)
                plsc.addupdate_scatter(hist_v, [b * L + st[k]["hist0"]], ones_i)
                ebm = plsc.load_gather(eb_v, [ebidx])
                plsc.store_scatter(out_v, [outidx + b],
                                   st[k]["near"] + ebm * st[k]["fnd"])
                nxt.append((cumidx + L, cdfidx + L, ebidx + 1, outidx + 1,
                            mpad + padstep[k]))
            return tuple(nxt)

        @plsc.parallel_loop(0, NB, unroll=4, carry=tuple(
            (s["hist0"], zeros_i) for s in st))
        def cpair(j, carry):
            uj = (j.astype(jnp.float32) * 2.0 + 1.0) * jnp.float32(0.5 / NB)
            nxt = []
            for k in (0, 1):
                histidx, ind = carry[k]
                s = st[k]
                h = plsc.load_gather(hist_v, [histidx])
                plsc.store_scatter(hist_v, [histidx], zeros_i)
                ind = ind + h
                g0 = ind - 1
                g1 = jnp.minimum(ind, N)
                cdf0 = plsc.load_gather(cdf_v, [g0 * L + s["cdf0"]])
                cdf1 = plsc.load_gather(cdf_v, [g1 * L + s["cdf0"]])
                eb0 = plsc.load_gather(eb_v, [s["ebbase"] + g0])
                eb1 = plsc.load_gather(eb_v, [s["ebbase"] + g1])
                d = jnp.maximum(cdf1 - cdf0, 1e-30)
                t = jnp.clip((uj - cdf0) / d, 0.0, 1.0)
                bj = eb0 + t * (eb1 - eb0)
                plsc.store_scatter(out_v, [s["outbase"] + ind + j],
                                   s["near"] + bj * s["fnd"])
                nxt.append((histidx + L, ind))
            return tuple(nxt)
        for k in (0, 1):
            plsc.store_scatter(hist_v, [NB * L + st[k]["hist0"]], zeros_i)

    tile0 = wid * RPT

    def in_copies(c, w_v, eb_v, sem):
        base = tile0 + c * CH
        return (
            pltpu.make_async_copy(w_hbm.at[pl.ds(base * N, CH * N)], w_v, sem),
            pltpu.make_async_copy(eb_hbm.at[pl.ds(base * NB, CH * NB)], eb_v, sem),
        )

    def out_copy(c, out_v, sem):
        base = tile0 + c * CH
        return pltpu.make_async_copy(
            out_v, out_hbm.at[pl.ds(base * OUT, CH * OUT)], sem)

    def compute(c, w_v, eb_v, out_v):
        def grp(gp, _):
            do_group_pair(c, gp, w_v, eb_v, out_v)
            return 0
        lax.fori_loop(0, NG // 2, grp, 0)

    pltpu.sync_copy(near_hbm.at[pl.ds(tile0, RPT)], near_v)
    pltpu.sync_copy(far_hbm.at[pl.ds(tile0, RPT)], far_v)

    for cp in in_copies(0, w_v0, eb_v0, sem_in0):
        cp.start()

    def do_pair(p, _):
        c0 = 2 * p
        c1 = c0 + 1
        for cp in in_copies(c1, w_v1, eb_v1, sem_in1):
            cp.start()

        @pl.when(p > 0)
        def _():
            out_copy(c0 - 2, out_v0, sem_out0).wait()
        for cp in in_copies(c0, w_v0, eb_v0, sem_in0):
            cp.wait()
        compute(c0, w_v0, eb_v0, out_v0)
        out_copy(c0, out_v0, sem_out0).start()

        @pl.when(p < NPAIR - 1)
        def _():
            for cp in in_copies(c0 + 2, w_v0, eb_v0, sem_in0):
                cp.start()

        @pl.when(p > 0)
        def _():
            out_copy(c1 - 2, out_v1, sem_out1).wait()
        for cp in in_copies(c1, w_v1, eb_v1, sem_in1):
            cp.wait()
        compute(c1, w_v1, eb_v1, out_v1)
        out_copy(c1, out_v1, sem_out1).start()
        return 0

    lax.fori_loop(0, NPAIR, do_pair, 0)
    out_copy(NCHUNK - 2, out_v0, sem_out0).wait()
    out_copy(NCHUNK - 1, out_v1, sem_out1).wait()


@jax.jit
def _run(w_flat, eb_flat, near_flat, far_flat):
    mesh = plsc.VectorSubcoreMesh(core_axis_name="c", subcore_axis_name="s")
    f = functools.partial(
        pl.kernel,
        out_type=jax.ShapeDtypeStruct((R * OUT,), jnp.float32),
        mesh=mesh,
        scratch_types=[
            pltpu.VMEM((CH * N,), jnp.float32),
            pltpu.VMEM((CH * N,), jnp.float32),
            pltpu.VMEM((CH * NB,), jnp.float32),
            pltpu.VMEM((CH * NB,), jnp.float32),
            pltpu.VMEM((RPT,), jnp.float32),
            pltpu.VMEM((RPT,), jnp.float32),
            pltpu.VMEM((2 * L * N,), jnp.float32),
            pltpu.VMEM((2 * L * NB,), jnp.float32),
            pltpu.VMEM((2 * L * (NB + 1),), jnp.int32),
            pltpu.VMEM((CH * OUT,), jnp.float32),
            pltpu.VMEM((CH * OUT,), jnp.float32),
            pltpu.SemaphoreType.DMA,
            pltpu.SemaphoreType.DMA,
            pltpu.SemaphoreType.DMA,
            pltpu.SemaphoreType.DMA,
        ],
        compiler_params=pltpu.CompilerParams(needs_layout_passes=False),
    )(_sc_body)
    return f(w_flat, eb_flat, near_flat, far_flat)


def kernel(weights, existing_bins, nears, fars):
    w_flat = weights.reshape(R * N)
    eb_flat = existing_bins.reshape(R * NB)
    near_flat = nears.reshape(R)
    far_flat = fars.reshape(R)
    out = _run(w_flat, eb_flat, near_flat, far_flat)
    return out.reshape(R, OUT)


# speedup vs baseline: 1.1369x; 1.1369x over previous
import functools

import jax
import jax.numpy as jnp
from jax import lax
from jax.experimental import pallas as pl
from jax.experimental.pallas import tpu as pltpu
from jax.experimental.pallas import tpu_sc as plsc

R = 65536
N = 128
NB = N + 1
OUT = 2 * NB
HIST_PAD = 0.01
EPS = 1e-5

NC = 2
NS = 16
L = 16
NW = NC * NS
RPT = R // NW
CH = 64
NG = CH // L
NCHUNK = RPT // CH
NPAIR = NCHUNK // 2


def _sc_body(w_hbm, eb_hbm, near_hbm, far_hbm, out_hbm,
             w_v0, w_v1, eb_v0, eb_v1, near_v, far_v, wT_v, cumw_v, cdf_v,
             hist_v, out_v0, out_v1, sem_in0, sem_in1, sem_out0, sem_out1):
    cid = lax.axis_index("c")
    sid = lax.axis_index("s")
    wid = sid * NC + cid
    lane = lax.iota(jnp.int32, L)
    ones_i = jnp.ones((L,), jnp.int32)
    zeros_i = jnp.zeros((L,), jnp.int32)
    zeros_f = jnp.zeros((L,), jnp.float32)

    @plsc.parallel_loop(0, NB + 1, unroll=8, carry=lane)
    def _z(i, idx):
        plsc.store_scatter(hist_v, [idx], zeros_i)
        return idx + L

    def do_group(c, g, w_v, eb_v, out_v):
        gl = g * L + lane
        ebbase = gl * NB
        outbase = gl * OUT
        nfidx = c * CH + gl
        near = plsc.load_gather(near_v, [nfidx])
        fnd = plsc.load_gather(far_v, [nfidx]) - near

        @plsc.parallel_loop(0, L, unroll=2,
                            carry=(g * (L * N) + lane, lane * 17))
        def _rp(r, carry):
            src0, dst0 = carry
            for i in range(N // L):
                v = plsc.load_gather(w_v, [src0 + i * L])
                plsc.store_scatter(wT_v, [dst0 + i * (L * 17)], v)
            return (src0 + N, dst0 + 1)

        @plsc.parallel_loop(0, N, unroll=8, carry=(lane, lane, zeros_f))
        def acarry(m, carry):
            widx, cumidx, cum = carry
            cum = cum + (plsc.load_gather(wT_v, [widx]) + HIST_PAD)
            plsc.store_scatter(cumw_v, [cumidx], cum)
            return (widx + 17, cumidx + L, cum)
        _, _, s = acarry
        pad = jnp.maximum(EPS - s, 0.0)
        padstep = pad * (1.0 / N)
        inv = 1.0 / (s + pad)

        plsc.store_scatter(cdf_v, [lane], zeros_f)
        plsc.addupdate_scatter(hist_v, [lane], ones_i)
        eb0v = plsc.load_gather(eb_v, [ebbase])
        plsc.store_scatter(out_v, [outbase], near + eb0v * fnd)

        @plsc.parallel_loop(1, NB, unroll=8,
                            carry=(lane, L + lane, ebbase + 1, outbase + 1,
                                   padstep))
        def _b(m, carry):
            cumidx, cdfidx, ebidx, outidx, mpad = carry
            cw = plsc.load_gather(cumw_v, [cumidx])
            c = jnp.minimum(1.0, (cw + mpad) * inv)
            plsc.store_scatter(cdf_v, [cdfidx], c)
            x = c * jnp.float32(NB) - 0.5
            xi = x.astype(jnp.int32)
            b = xi + jnp.where(xi.astype(jnp.float32) < x, ones_i, zeros_i)
            plsc.addupdate_scatter(hist_v, [b * L + lane], ones_i)
            ebm = plsc.load_gather(eb_v, [ebidx])
            plsc.store_scatter(out_v, [outidx + b], near + ebm * fnd)
            return (cumidx + L, cdfidx + L, ebidx + 1, outidx + 1,
                    mpad + padstep)

        @plsc.parallel_loop(0, NB, unroll=8, carry=(lane, zeros_i))
        def _c(j, carry):
            histidx, ind = carry
            h = plsc.load_gather(hist_v, [histidx])
            plsc.store_scatter(hist_v, [histidx], zeros_i)
            ind = ind + h
            g0 = ind - 1
            g1 = jnp.minimum(ind, N)
            cdf0 = plsc.load_gather(cdf_v, [g0 * L + lane])
            cdf1 = plsc.load_gather(cdf_v, [g1 * L + lane])
            eb0 = plsc.load_gather(eb_v, [ebbase + g0])
            eb1 = plsc.load_gather(eb_v, [ebbase + g1])
            uj = (j.astype(jnp.float32) * 2.0 + 1.0) * jnp.float32(0.5 / NB)
            d = jnp.maximum(cdf1 - cdf0, 1e-30)
            t = jnp.clip((uj - cdf0) / d, 0.0, 1.0)
            bj = eb0 + t * (eb1 - eb0)
            plsc.store_scatter(out_v, [outbase + ind + j], near + bj * fnd)
            return (histidx + L, ind)
        plsc.store_scatter(hist_v, [NB * L + lane], zeros_i)

    tile0 = wid * RPT

    def in_copies(c, w_v, eb_v, sem):
        base = tile0 + c * CH
        return (
            pltpu.make_async_copy(w_hbm.at[pl.ds(base * N, CH * N)], w_v, sem),
            pltpu.make_async_copy(eb_hbm.at[pl.ds(base * NB, CH * NB)], eb_v, sem),
        )

    def out_copy(c, out_v, sem):
        base = tile0 + c * CH
        return pltpu.make_async_copy(
            out_v, out_hbm.at[pl.ds(base * OUT, CH * OUT)], sem)

    def compute(c, w_v, eb_v, out_v):
        def grp(g, _):
            do_group(c, g, w_v, eb_v, out_v)
            return 0
        lax.fori_loop(0, NG, grp, 0)

    pltpu.sync_copy(near_hbm.at[pl.ds(tile0, RPT)], near_v)
    pltpu.sync_copy(far_hbm.at[pl.ds(tile0, RPT)], far_v)

    for cp in in_copies(0, w_v0, eb_v0, sem_in0):
        cp.start()

    def do_pair(p, _):
        c0 = 2 * p
        c1 = c0 + 1
        for cp in in_copies(c1, w_v1, eb_v1, sem_in1):
            cp.start()

        @pl.when(p > 0)
        def _():
            out_copy(c0 - 2, out_v0, sem_out0).wait()
        for cp in in_copies(c0, w_v0, eb_v0, sem_in0):
            cp.wait()
        compute(c0, w_v0, eb_v0, out_v0)
        out_copy(c0, out_v0, sem_out0).start()

        @pl.when(p < NPAIR - 1)
        def _():
            for cp in in_copies(c0 + 2, w_v0, eb_v0, sem_in0):
                cp.start()

        @pl.when(p > 0)
        def _():
            out_copy(c1 - 2, out_v1, sem_out1).wait()
        for cp in in_copies(c1, w_v1, eb_v1, sem_in1):
            cp.wait()
        compute(c1, w_v1, eb_v1, out_v1)
        out_copy(c1, out_v1, sem_out1).start()
        return 0

    lax.fori_loop(0, NPAIR, do_pair, 0)
    out_copy(NCHUNK - 2, out_v0, sem_out0).wait()
    out_copy(NCHUNK - 1, out_v1, sem_out1).wait()


@jax.jit
def _run(w_flat, eb_flat, near_flat, far_flat):
    mesh = plsc.VectorSubcoreMesh(core_axis_name="c", subcore_axis_name="s")
    f = functools.partial(
        pl.kernel,
        out_type=jax.ShapeDtypeStruct((R * OUT,), jnp.float32),
        mesh=mesh,
        scratch_types=[
            pltpu.VMEM((CH * N,), jnp.float32),
            pltpu.VMEM((CH * N,), jnp.float32),
            pltpu.VMEM((CH * NB,), jnp.float32),
            pltpu.VMEM((CH * NB,), jnp.float32),
            pltpu.VMEM((RPT,), jnp.float32),
            pltpu.VMEM((RPT,), jnp.float32),
            pltpu.VMEM((N * 17,), jnp.float32),
            pltpu.VMEM((L * N,), jnp.float32),
            pltpu.VMEM((L * NB,), jnp.float32),
            pltpu.VMEM((L * (NB + 1),), jnp.int32),
            pltpu.VMEM((CH * OUT,), jnp.float32),
            pltpu.VMEM((CH * OUT,), jnp.float32),
            pltpu.SemaphoreType.DMA,
            pltpu.SemaphoreType.DMA,
            pltpu.SemaphoreType.DMA,
            pltpu.SemaphoreType.DMA,
        ],
        compiler_params=pltpu.CompilerParams(needs_layout_passes=False),
    )(_sc_body)
    return f(w_flat, eb_flat, near_flat, far_flat)


def kernel(weights, existing_bins, nears, fars):
    w_flat = weights.reshape(R * N)
    eb_flat = existing_bins.reshape(R * NB)
    near_flat = nears.reshape(R)
    far_flat = fars.reshape(R)
    out = _run(w_flat, eb_flat, near_flat, far_flat)
    return out.reshape(R, OUT)
